# baseline (speedup 1.0000x reference)
import functools

import jax
import jax.numpy as jnp
from jax import lax
from jax.experimental import pallas as pl
from jax.experimental.pallas import tpu as pltpu
from jax.experimental.pallas import tpu_sc as plsc

_MIN_SIZE = 0.01
_MAX_SIZE = 0.999
_NW = 32
_CHR = 4000
_ROWS = 1_000_000
_NCHUNK = _ROWS // _CHR
_BASE_CHUNKS = _NCHUNK // _NW
_EXTRA = _NCHUNK % _NW


def _normalize_chunk(buf, sv):
    lanes = lax.iota(jnp.int32, 16)
    zeros = lanes * 0
    ones = zeros + 1

    def blk(i, carry):
        ri = lanes + i * 16
        a = plsc.load_gather(buf, [ri, zeros])
        b = plsc.load_gather(buf, [ri, ones])
        t = a * a + b * b
        bits = plsc.bitcast(t, jnp.int32)
        bits = 0x5F3759DF - lax.shift_right_logical(bits, 1)
        y = plsc.bitcast(bits, jnp.float32)
        y = y * (1.5 - 0.5 * t * y * y)
        y = y * (1.5 - 0.5 * t * y * y)
        norm = t * y
        factor = jnp.where(norm >= 1e-12, sv * y, sv * 1e12)
        plsc.store_scatter(buf, [ri, zeros], a * factor)
        plsc.store_scatter(buf, [ri, ones], b * factor)
        return carry

    lax.fori_loop(0, _CHR // 16, blk, 0)


def _make_sc_call():
    mesh = plsc.VectorSubcoreMesh(core_axis_name="c", subcore_axis_name="s")

    @functools.partial(
        pl.kernel,
        out_type=jax.ShapeDtypeStruct((_ROWS, 2), jnp.float32),
        mesh=mesh,
        scratch_types=[
            pltpu.VMEM((_CHR, 2), jnp.float32),
            pltpu.VMEM((16,), jnp.float32),
        ],
        compiler_params=pltpu.CompilerParams(
            needs_layout_passes=False, use_tc_tiling_on_sc=False
        ),
    )
    def run(w_hbm, s_hbm, out_hbm, buf, sbuf):
        wid = lax.axis_index("s") * 2 + lax.axis_index("c")
        pltpu.sync_copy(s_hbm, sbuf)
        sv = jnp.clip(sbuf[...], _MIN_SIZE, _MAX_SIZE)
        nchunks = jnp.where(wid < _EXTRA, _BASE_CHUNKS + 1, _BASE_CHUNKS)

        def chunk_body(j, carry):
            cid = j * _NW + wid
            off = pl.multiple_of(cid * _CHR, _CHR)
            pltpu.sync_copy(w_hbm.at[pl.ds(off, _CHR)], buf)
            _normalize_chunk(buf, sv)
            pltpu.sync_copy(buf, out_hbm.at[pl.ds(off, _CHR)])
            return carry

        lax.fori_loop(0, nchunks, chunk_body, 0)

    return run


_sc_call = _make_sc_call()


def kernel(weight, scale):
    s16 = jnp.broadcast_to(scale, (16,))
    return _sc_call(weight, s16)


# speedup vs baseline: 21.8746x; 21.8746x over previous
import functools

import jax
import jax.numpy as jnp
from jax import lax
from jax.experimental import pallas as pl
from jax.experimental.pallas import tpu as pltpu
from jax.experimental.pallas import tpu_sc as plsc

_MIN_SIZE = 0.01
_MAX_SIZE = 0.999
_NW = 32
_CR = 4000
_ROWS = 1_000_000
_NCHUNK = _ROWS // _CR
_BASE_CHUNKS = _NCHUNK // _NW
_EXTRA = _NCHUNK % _NW


def _normalize_chunk(bufa, bufb, sv):
    def blk(i, carry):
        ds = pl.ds(i * 16, 16)
        a = bufa[ds]
        b = bufb[ds]
        t = a * a + b * b
        bits = plsc.bitcast(t, jnp.int32)
        bits = 0x5F3759DF - lax.shift_right_logical(bits, 1)
        y = plsc.bitcast(bits, jnp.float32)
        y = y * (1.5 - 0.5 * t * y * y)
        y = y * (1.5 - 0.5 * t * y * y)
        norm = t * y
        factor = jnp.where(norm >= 1e-12, sv * y, sv * 1e12)
        bufa[ds] = a * factor
        bufb[ds] = b * factor
        return carry

    lax.fori_loop(0, _CR // 16, blk, 0)


def _make_sc_call():
    mesh = plsc.VectorSubcoreMesh(core_axis_name="c", subcore_axis_name="s")

    @functools.partial(
        pl.kernel,
        out_type=jax.ShapeDtypeStruct((2 * _ROWS,), jnp.float32),
        mesh=mesh,
        scratch_types=[
            pltpu.VMEM((_CR,), jnp.float32),
            pltpu.VMEM((_CR,), jnp.float32),
            pltpu.VMEM((16,), jnp.float32),
        ],
        compiler_params=pltpu.CompilerParams(
            needs_layout_passes=False, use_tc_tiling_on_sc=False
        ),
    )
    def run(w_hbm, s_hbm, out_hbm, bufa, bufb, sbuf):
        wid = lax.axis_index("s") * 2 + lax.axis_index("c")
        pltpu.sync_copy(s_hbm, sbuf)
        sv = jnp.clip(sbuf[...], _MIN_SIZE, _MAX_SIZE)
        nchunks = jnp.where(wid < _EXTRA, _BASE_CHUNKS + 1, _BASE_CHUNKS)

        def chunk_body(j, carry):
            cid = j * _NW + wid
            offa = pl.multiple_of(cid * _CR, _CR)
            offb = pl.multiple_of(_ROWS + cid * _CR, _CR)
            pltpu.sync_copy(w_hbm.at[pl.ds(offa, _CR)], bufa)
            pltpu.sync_copy(w_hbm.at[pl.ds(offb, _CR)], bufb)
            _normalize_chunk(bufa, bufb, sv)
            pltpu.sync_copy(bufa, out_hbm.at[pl.ds(offa, _CR)])
            pltpu.sync_copy(bufb, out_hbm.at[pl.ds(offb, _CR)])
            return carry

        lax.fori_loop(0, nchunks, chunk_body, 0)

    return run


_sc_call = _make_sc_call()


def kernel(weight, scale):
    s16 = jnp.broadcast_to(scale, (16,))
    flat = weight.T.reshape(2 * _ROWS)
    out = _sc_call(flat, s16)
    return out.reshape(2, _ROWS).T


# speedup vs baseline: 40.0864x; 1.8326x over previous
import functools

import jax
import jax.numpy as jnp
from jax import lax
from jax.experimental import pallas as pl
from jax.experimental.pallas import tpu as pltpu
from jax.experimental.pallas import tpu_sc as plsc

_MIN_SIZE = 0.01
_MAX_SIZE = 0.999
_NW = 32
_CR = 4000
_ROWS = 1_000_000
_NCHUNK = _ROWS // _CR
_MAXJ = -(-_NCHUNK // _NW)
_UNROLL = 5


def _normalize_chunk(bufa, bufb, sv):
    f_tiny = sv * 1e12

    def blk(i, carry):
        for u in range(_UNROLL):
            ds = pl.ds((i * _UNROLL + u) * 16, 16)
            a = bufa[ds]
            b = bufb[ds]
            t = a * a + b * b
            th = 0.5 * t
            bits = plsc.bitcast(t, jnp.int32)
            bits = 0x5F3759DF - lax.shift_right_logical(bits, 1)
            y = plsc.bitcast(bits, jnp.float32)
            y = y * (1.5 - th * (y * y))
            y = y * (1.5 - th * (y * y))
            factor = jnp.where(t >= 1e-24, sv * y, f_tiny)
            bufa[ds] = a * factor
            bufb[ds] = b * factor
        return carry

    lax.fori_loop(0, _CR // (16 * _UNROLL), blk, 0)


def _make_sc_call():
    mesh = plsc.VectorSubcoreMesh(core_axis_name="c", subcore_axis_name="s")

    _B = 3

    @functools.partial(
        pl.kernel,
        out_type=jax.ShapeDtypeStruct((2 * _ROWS,), jnp.float32),
        mesh=mesh,
        scratch_types=(
            [pltpu.VMEM((_CR,), jnp.float32)] * (2 * _B)
            + [pltpu.VMEM((16,), jnp.float32)]
            + [pltpu.SemaphoreType.DMA] * (2 * _B)
        ),
        compiler_params=pltpu.CompilerParams(
            needs_layout_passes=False, use_tc_tiling_on_sc=False
        ),
    )
    def run(w_hbm, s_hbm, out_hbm, *scr):
        abuf = scr[0:_B]
        bbuf = scr[_B : 2 * _B]
        sbuf = scr[2 * _B]
        si = scr[2 * _B + 1 : 3 * _B + 1]
        so = scr[3 * _B + 1 : 4 * _B + 1]

        wid = lax.axis_index("s") * 2 + lax.axis_index("c")
        pltpu.sync_copy(s_hbm, sbuf)
        sv = jnp.clip(sbuf[...], _MIN_SIZE, _MAX_SIZE)
        base = wid * _CR

        def offs(j):
            offa = pl.multiple_of(base + j * _NW * _CR, _CR)
            offb = pl.multiple_of(_ROWS + base + j * _NW * _CR, _CR)
            return offa, offb

        def start_in(j):
            offa, offb = offs(j)
            p = j % _B
            pltpu.async_copy(w_hbm.at[pl.ds(offa, _CR)], abuf[p], si[p])
            pltpu.async_copy(w_hbm.at[pl.ds(offb, _CR)], bbuf[p], si[p])

        def wait_in(j):
            p = j % _B
            pltpu.make_async_copy(w_hbm.at[pl.ds(0, _CR)], abuf[p], si[p]).wait()
            pltpu.make_async_copy(w_hbm.at[pl.ds(0, _CR)], bbuf[p], si[p]).wait()

        def start_out(j):
            offa, offb = offs(j)
            p = j % _B
            pltpu.async_copy(abuf[p], out_hbm.at[pl.ds(offa, _CR)], so[p])
            pltpu.async_copy(bbuf[p], out_hbm.at[pl.ds(offb, _CR)], so[p])

        def wait_out(j):
            p = j % _B
            pltpu.make_async_copy(abuf[p], out_hbm.at[pl.ds(0, _CR)], so[p]).wait()
            pltpu.make_async_copy(bbuf[p], out_hbm.at[pl.ds(0, _CR)], so[p]).wait()

        def valid(j):
            return j * _NW + wid < _NCHUNK

        start_in(0)
        if _MAXJ > 1:
            start_in(1)
        for j in range(_MAXJ):
            if j + 2 < _MAXJ:
                if j - 1 >= 0:
                    wait_out(j - 1)

                @pl.when(valid(j + 2))
                def _():
                    start_in(j + 2)

            if j < _MAXJ - 1:
                wait_in(j)
                _normalize_chunk(abuf[j % _B], bbuf[j % _B], sv)
                start_out(j)
            else:

                @pl.when(valid(j))
                def _():
                    wait_in(j)
                    _normalize_chunk(abuf[j % _B], bbuf[j % _B], sv)
                    start_out(j)

        wait_out(_MAXJ - 3)
        wait_out(_MAXJ - 2)

        @pl.when(valid(_MAXJ - 1))
        def _():
            wait_out(_MAXJ - 1)

    return run


_sc_call = _make_sc_call()


def kernel(weight, scale):
    s16 = jnp.broadcast_to(scale, (16,))
    flat = weight.T.reshape(2 * _ROWS)
    out = _sc_call(flat, s16)
    return out.reshape(2, _ROWS).T
